# baseline (device time: 26880 ns/iter reference)
import jax
import jax.numpy as jnp
from jax import lax
from jax.experimental import pallas as pl
from jax.experimental.pallas import tpu as pltpu

N_DEV = 16
LOG2_N = 4
B, SQ, SKV, HQ_SH, DH = 2, 128, 128, 4, 64
D_MODEL = 512


def kernel(x, Wq, K_ext, V_ext, Wo):
    my_i = lax.axis_index("i")
    K_sh = lax.dynamic_slice_in_dim(K_ext, my_i * HQ_SH, HQ_SH, axis=2)
    V_sh = lax.dynamic_slice_in_dim(V_ext, my_i * HQ_SH, HQ_SH, axis=2)
    K_sh = jnp.transpose(K_sh.astype(jnp.bfloat16), (0, 2, 1, 3))
    V_sh = jnp.transpose(V_sh.astype(jnp.bfloat16), (0, 2, 1, 3))
    x16 = x.astype(jnp.bfloat16)
    Wq16 = Wq.astype(jnp.bfloat16)
    Wo16 = Wo.astype(jnp.bfloat16)

    def body(x_ref, wq_ref, k_ref, v_ref, wo_ref, out_ref,
             ctx_ref, acc_ref, send_ref, recv_ref, send_sems, recv_sems):
        me = lax.axis_index("i")
        partners = [me ^ (1 << j) for j in range(LOG2_N)]

        def bit(p, c):
            return p if c == 0 else LOG2_N - 1 - p

        def make_rdma(p, c):
            return pltpu.make_async_remote_copy(
                src_ref=send_ref.at[p, c],
                dst_ref=recv_ref.at[p, c],
                send_sem=send_sems.at[p, c],
                recv_sem=recv_sems.at[p, c],
                device_id=(partners[bit(p, c)],),
                device_id_type=pl.DeviceIdType.MESH,
            )

        barrier = pltpu.get_barrier_semaphore()
        for j in range(LOG2_N):
            pl.semaphore_signal(barrier, inc=1, device_id=(partners[j],),
                                device_id_type=pl.DeviceIdType.MESH)
        pl.semaphore_wait(barrier, LOG2_N)

        wq = wq_ref[...]
        wo = wo_ref[...]
        ri = lax.broadcasted_iota(jnp.int32, (SQ, SKV), 0)
        ci = lax.broadcasted_iota(jnp.int32, (SQ, SKV), 1)
        qb = ri // 64
        kb = ci // 64
        mask = (qb == kb) | ((kb % 4) == (qb % 4))
        for b in range(B):
            xb = x_ref[b]
            q_all = jnp.dot(xb, wq, preferred_element_type=jnp.float32)
            q_all = q_all.astype(jnp.bfloat16)
            for h in range(HQ_SH):
                q = q_all[:, h * DH:(h + 1) * DH]
                k = k_ref[b, h]
                s = lax.dot_general(q, k, (((1,), (1,)), ((), ())),
                                    preferred_element_type=jnp.float32)
                s = s * 0.125
                s = jnp.where(mask, s, -1e9)
                m = jnp.max(s, axis=-1, keepdims=True)
                w = jnp.exp(s - m)
                w = w / jnp.sum(w, axis=-1, keepdims=True)
                v = v_ref[b, h]
                ctx = jnp.dot(w.astype(jnp.bfloat16), v,
                              preferred_element_type=jnp.float32)
                ctx_ref[b, :, h * DH:(h + 1) * DH] = ctx.astype(jnp.bfloat16)
            part = jnp.dot(ctx_ref[b], wo,
                           preferred_element_type=jnp.float32)
            acc_ref[b] = part
            send_ref[0, b] = part.astype(jnp.bfloat16)
            make_rdma(0, b).start()

        for p in range(LOG2_N):
            for c in range(B):
                make_rdma(p, c).wait_recv()
                s = acc_ref[c] + recv_ref[p, c].astype(jnp.float32)
                if p + 1 < LOG2_N:
                    acc_ref[c] = s
                    send_ref[p + 1, c] = s.astype(jnp.bfloat16)
                    make_rdma(p + 1, c).start()
                else:
                    out_ref[c] = s.astype(jnp.bfloat16)

        for p in range(LOG2_N):
            for c in range(B):
                make_rdma(p, c).wait_send()

    return pl.pallas_call(
        body,
        out_shape=jax.ShapeDtypeStruct((B, SQ, D_MODEL), jnp.bfloat16),
        in_specs=[pl.BlockSpec(memory_space=pltpu.VMEM)] * 5,
        out_specs=pl.BlockSpec(memory_space=pltpu.VMEM),
        scratch_shapes=[
            pltpu.VMEM((B, SQ, HQ_SH * DH), jnp.bfloat16),
            pltpu.VMEM((B, SQ, D_MODEL), jnp.float32),
            pltpu.VMEM((LOG2_N, B, SQ, D_MODEL), jnp.bfloat16),
            pltpu.VMEM((LOG2_N, B, SQ, D_MODEL), jnp.bfloat16),
            pltpu.SemaphoreType.DMA((LOG2_N, B)),
            pltpu.SemaphoreType.DMA((LOG2_N, B)),
        ],
        compiler_params=pltpu.CompilerParams(collective_id=0),
    )(x16, Wq16, K_sh, V_sh, Wo16)
